# Initial kernel scaffold; baseline (speedup 1.0000x reference)
import jax
import jax.numpy as jnp
from jax.experimental import pallas as pl


def kernel(x, edge_index, edge_attr, edge_sh, W1, b1, W2, b2):
    raise NotImplementedError("write your pallas kernel here")


# speedup vs baseline: 1.5213x; 1.5213x over previous
import functools

import jax
import jax.numpy as jnp
from jax import lax
from jax.experimental import pallas as pl
from jax.experimental.pallas import tpu as pltpu
from jax.experimental.pallas import tpu_sc as plsc

NS = 64
NV = 16
HID = 192
N_NODES = 10000
N_EDGES = 160000

WEXT = NS + 3 * NV
MROW = 128
NTILES = 32
CHUNK = 128
ITERS = 40
E_PAD = NTILES * ITERS * CHUNK
ACC_ROWS = 10240
DUMMY = N_NODES
ROWS_PER_TILE = ACC_ROWS // 16
MLP_BLK = 1024


def _mlp_body(a_ref, sh_ref, w1_ref, b1_ref, w2_ref, b2_ref, o_ref):
    h = jnp.maximum(
        jnp.dot(a_ref[...], w1_ref[...], preferred_element_type=jnp.float32)
        + b1_ref[...], 0.0)
    w = jnp.dot(h, w2_ref[...], preferred_element_type=jnp.float32) + b2_ref[...]
    ws = w[:, :NS]
    wv = w[:, NS:NS + NV]
    u0 = wv * sh_ref[:, 1:2]
    u1 = wv * sh_ref[:, 2:3]
    u2 = wv * sh_ref[:, 3:4]
    o_ref[...] = jnp.concatenate([ws, u0, u1, u2], axis=1)


def _mlp(edge_attr, edge_sh, W1, b1, W2, b2):
    grid = (E_PAD // MLP_BLK,)
    return pl.pallas_call(
        _mlp_body,
        grid=grid,
        in_specs=[
            pl.BlockSpec((MLP_BLK, HID), lambda i: (i, 0)),
            pl.BlockSpec((MLP_BLK, 4), lambda i: (i, 0)),
            pl.BlockSpec((HID, HID), lambda i: (0, 0)),
            pl.BlockSpec((1, HID), lambda i: (0, 0)),
            pl.BlockSpec((HID, NS + NV), lambda i: (0, 0)),
            pl.BlockSpec((1, NS + NV), lambda i: (0, 0)),
        ],
        out_specs=pl.BlockSpec((MLP_BLK, WEXT), lambda i: (i, 0)),
        out_shape=jax.ShapeDtypeStruct((E_PAD, WEXT), jnp.float32),
    )(edge_attr, edge_sh, W1, b1, W2, b2)


def _sc_kernel(x, src, dst, wext_flat):
    mesh = plsc.VectorSubcoreMesh(core_axis_name="c", subcore_axis_name="s")

    @functools.partial(
        pl.kernel,
        mesh=mesh,
        out_type=jax.ShapeDtypeStruct((2, ACC_ROWS, MROW), jnp.float32),
        scratch_types=[
            pltpu.VMEM((CHUNK,), jnp.int32),
            pltpu.VMEM((CHUNK,), jnp.int32),
            pltpu.VMEM((CHUNK, 128), jnp.float32),
            pltpu.VMEM((CHUNK * WEXT,), jnp.float32),
            pltpu.VMEM((CHUNK, MROW), jnp.float32),
            pltpu.VMEM_SHARED((ACC_ROWS, MROW), jnp.float32),
            pltpu.SemaphoreType.DMA,
        ],
    )
    def body(x_hbm, src_hbm, dst_hbm, w_hbm, out_hbm,
             idxs_v, idxd_v, hsrc_v, w_v, msg_v, acc_sh, sem):
        cid = lax.axis_index("c")
        sid = lax.axis_index("s")

        zeros16 = jnp.zeros((16,), jnp.float32)

        def zrow(i, _):
            r = i // (MROW // 16)
            c = i % (MROW // 16)
            msg_v[r, pl.ds(c * 16, 16)] = zeros16
            return 0

        lax.fori_loop(0, CHUNK * (MROW // 16), zrow, 0)

        for k in range(ROWS_PER_TILE // CHUNK):
            pltpu.sync_copy(
                msg_v, acc_sh.at[pl.ds(sid * ROWS_PER_TILE + k * CHUNK, CHUNK)])

        ii = lax.broadcasted_iota(jnp.int32, (16,), 0)
        cvec = jnp.where(ii == 0, 1.0, 0.0).astype(jnp.float32)

        def crow(r, _):
            msg_v[r, pl.ds(WEXT, 16)] = cvec
            return 0

        lax.fori_loop(0, CHUNK, crow, 0)

        plsc.subcore_barrier()

        tbase = (cid * 16 + sid) * (ITERS * CHUNK)

        def chunk_body(it, _):
            base = tbase + it * CHUNK
            pltpu.sync_copy(src_hbm.at[pl.ds(base, CHUNK)], idxs_v)
            pltpu.sync_copy(dst_hbm.at[pl.ds(base, CHUNK)], idxd_v)
            pltpu.sync_copy(w_hbm.at[pl.ds(base * WEXT, CHUNK * WEXT)], w_v)
            pltpu.async_copy(x_hbm.at[idxs_v], hsrc_v, sem).wait()

            def edge_body(e, _):
                wb = e * WEXT
                h0 = hsrc_v[e, pl.ds(0, 16)]
                msg_v[e, pl.ds(0, 16)] = h0 * w_v[pl.ds(wb, 16)]
                for k in range(1, NS // 16):
                    msg_v[e, pl.ds(k * 16, 16)] = (
                        hsrc_v[e, pl.ds(k * 16, 16)] * w_v[pl.ds(wb + k * 16, 16)])
                for j in range(3):
                    off = NS + j * 16
                    msg_v[e, pl.ds(off, 16)] = h0 * w_v[pl.ds(wb + off, 16)]
                return 0

            lax.fori_loop(0, CHUNK, edge_body, 0)

            pltpu.sync_copy(msg_v, acc_sh.at[idxd_v], add=True)
            return 0

        lax.fori_loop(0, ITERS, chunk_body, 0)

        plsc.subcore_barrier()

        pltpu.sync_copy(
            acc_sh.at[pl.ds(sid * ROWS_PER_TILE, ROWS_PER_TILE)],
            out_hbm.at[cid, pl.ds(sid * ROWS_PER_TILE, ROWS_PER_TILE)])

    return body(x, src, dst, wext_flat)


def _combine_body(p_ref, o_ref):
    a = p_ref[0] + p_ref[1]
    deg = jnp.maximum(a[:, WEXT:WEXT + 1], 1.0)
    s = a[:, :NS] / deg
    v = a[:, NS:WEXT] / deg
    r = lax.broadcasted_iota(jnp.int32, (3 * NV, 3 * NV), 0)
    c = lax.broadcasted_iota(jnp.int32, (3 * NV, 3 * NV), 1)
    perm = ((c % 3) * NV + (c // 3) == r).astype(jnp.float32)
    vp = jnp.dot(v, perm, preferred_element_type=jnp.float32)
    o_ref[...] = jnp.concatenate([s, vp], axis=1)


def _combine(partials):
    grid = (10,)
    blk = N_NODES // 10
    return pl.pallas_call(
        _combine_body,
        grid=grid,
        in_specs=[pl.BlockSpec((2, blk, MROW), lambda i: (0, i, 0))],
        out_specs=pl.BlockSpec((blk, NS + 3 * NV), lambda i: (i, 0)),
        out_shape=jax.ShapeDtypeStruct((N_NODES, NS + 3 * NV), jnp.float32),
    )(partials)


def kernel(x, edge_index, edge_attr, edge_sh, W1, b1, W2, b2):
    pad = E_PAD - N_EDGES
    src = jnp.pad(edge_index[0].astype(jnp.int32), (0, pad))
    dst = jnp.pad(edge_index[1].astype(jnp.int32), (0, pad),
                  constant_values=DUMMY)
    ea = jnp.pad(edge_attr, ((0, pad), (0, 0)))
    sh = jnp.pad(edge_sh, ((0, pad), (0, 0)))

    wext = _mlp(ea, sh, W1, b1.reshape(1, HID), W2, b2.reshape(1, NS + NV))
    x_pad = jnp.pad(x, ((0, 0), (0, 128 - NS)))
    partials = _sc_kernel(x_pad, src, dst, wext.reshape(-1))
    return _combine(partials)


# speedup vs baseline: 2.6844x; 1.7646x over previous
import functools

import jax
import jax.numpy as jnp
from jax import lax
from jax.experimental import pallas as pl
from jax.experimental.pallas import tpu as pltpu
from jax.experimental.pallas import tpu_sc as plsc

NS = 64
NV = 16
HID = 192
N_NODES = 10000
N_EDGES = 160000

WEXT = NS + 3 * NV
MROW = 128
NTILES = 32
CHUNK = 128
EDGES_PER_TILE = N_EDGES // NTILES
FULL_ITERS = EDGES_PER_TILE // CHUNK
TAIL = EDGES_PER_TILE - FULL_ITERS * CHUNK
ACC_ROWS = 10112
ROWS_PER_TILE = ACC_ROWS // 16
MLP_BLK = 1000


def _mlp_body(a_ref, sh_ref, w1_ref, b1_ref, w2_ref, b2_ref, o_ref):
    h = jnp.maximum(
        jnp.dot(a_ref[...], w1_ref[...], preferred_element_type=jnp.float32)
        + b1_ref[...], 0.0)
    w = jnp.dot(h, w2_ref[...], preferred_element_type=jnp.float32) + b2_ref[...]
    ws = w[:, :NS]
    wv = w[:, NS:NS + NV]
    u0 = wv * sh_ref[:, 1:2]
    u1 = wv * sh_ref[:, 2:3]
    u2 = wv * sh_ref[:, 3:4]
    pad = jnp.zeros((MLP_BLK, MROW - WEXT), jnp.float32)
    o_ref[...] = jnp.concatenate([ws, u0, u1, u2, pad], axis=1)


def _mlp(edge_attr, edge_sh, W1, b1, W2, b2):
    grid = (N_EDGES // MLP_BLK,)
    return pl.pallas_call(
        _mlp_body,
        grid=grid,
        in_specs=[
            pl.BlockSpec((MLP_BLK, HID), lambda i: (i, 0)),
            pl.BlockSpec((MLP_BLK, 4), lambda i: (i, 0)),
            pl.BlockSpec((HID, HID), lambda i: (0, 0)),
            pl.BlockSpec((1, HID), lambda i: (0, 0)),
            pl.BlockSpec((HID, NS + NV), lambda i: (0, 0)),
            pl.BlockSpec((1, NS + NV), lambda i: (0, 0)),
        ],
        out_specs=pl.BlockSpec((MLP_BLK, MROW), lambda i: (i, 0)),
        out_shape=jax.ShapeDtypeStruct((N_EDGES, MROW), jnp.float32),
    )(edge_attr, edge_sh, W1, b1, W2, b2)


def _sc_kernel(x, src, dst, wext):
    mesh = plsc.VectorSubcoreMesh(core_axis_name="c", subcore_axis_name="s")

    @functools.partial(
        pl.kernel,
        mesh=mesh,
        out_type=jax.ShapeDtypeStruct((2, ACC_ROWS, MROW), jnp.float32),
        scratch_types=[
            pltpu.VMEM((CHUNK,), jnp.int32),
            pltpu.VMEM((CHUNK,), jnp.int32),
            pltpu.VMEM((TAIL,), jnp.int32),
            pltpu.VMEM((TAIL,), jnp.int32),
            pltpu.VMEM((CHUNK, 128), jnp.float32),
            pltpu.VMEM((CHUNK, MROW), jnp.float32),
            pltpu.VMEM((CHUNK, MROW), jnp.float32),
            pltpu.VMEM_SHARED((ACC_ROWS, MROW), jnp.float32),
            pltpu.SemaphoreType.DMA,
        ],
    )
    def body(x_hbm, src_hbm, dst_hbm, w_hbm, out_hbm,
             idxs_v, idxd_v, idxs8_v, idxd8_v, hsrc_v, w_v, msg_v, acc_sh, sem):
        cid = lax.axis_index("c")
        sid = lax.axis_index("s")

        zeros16 = jnp.zeros((16,), jnp.float32)

        def zrow(i, _):
            r = i // (MROW // 16)
            c = i % (MROW // 16)
            msg_v[r, pl.ds(c * 16, 16)] = zeros16
            return 0

        lax.fori_loop(0, CHUNK * (MROW // 16), zrow, 0)

        for k in range(ROWS_PER_TILE // CHUNK):
            pltpu.sync_copy(
                msg_v, acc_sh.at[pl.ds(sid * ROWS_PER_TILE + k * CHUNK, CHUNK)])
        rem = ROWS_PER_TILE % CHUNK
        if rem:
            pltpu.sync_copy(
                msg_v.at[pl.ds(0, rem)],
                acc_sh.at[pl.ds(sid * ROWS_PER_TILE
                                + (ROWS_PER_TILE // CHUNK) * CHUNK, rem)])

        ii = lax.broadcasted_iota(jnp.int32, (16,), 0)
        cvec = jnp.where(ii == 0, 1.0, 0.0).astype(jnp.float32)

        def crow(r, _):
            msg_v[r, pl.ds(WEXT, 16)] = cvec
            return 0

        lax.fori_loop(0, CHUNK, crow, 0)

        plsc.subcore_barrier()

        tbase = (cid * 16 + sid) * EDGES_PER_TILE

        def edge_body(e, _):
            h0 = hsrc_v[e, pl.ds(0, 16)]
            msg_v[e, pl.ds(0, 16)] = h0 * w_v[e, pl.ds(0, 16)]
            for k in range(1, NS // 16):
                msg_v[e, pl.ds(k * 16, 16)] = (
                    hsrc_v[e, pl.ds(k * 16, 16)] * w_v[e, pl.ds(k * 16, 16)])
            for j in range(3):
                off = NS + j * 16
                msg_v[e, pl.ds(off, 16)] = h0 * w_v[e, pl.ds(off, 16)]
            return 0

        def chunk_body(it, _):
            base = tbase + it * CHUNK
            pltpu.sync_copy(src_hbm.at[pl.ds(base, CHUNK)], idxs_v)
            pltpu.sync_copy(dst_hbm.at[pl.ds(base, CHUNK)], idxd_v)
            pltpu.sync_copy(w_hbm.at[pl.ds(base, CHUNK)], w_v)
            pltpu.async_copy(x_hbm.at[idxs_v], hsrc_v, sem).wait()
            lax.fori_loop(0, CHUNK, edge_body, 0)
            pltpu.sync_copy(msg_v, acc_sh.at[idxd_v], add=True)
            return 0

        lax.fori_loop(0, FULL_ITERS, chunk_body, 0)

        tb = tbase + FULL_ITERS * CHUNK
        pltpu.sync_copy(src_hbm.at[pl.ds(tb, TAIL)], idxs8_v)
        pltpu.sync_copy(dst_hbm.at[pl.ds(tb, TAIL)], idxd8_v)
        pltpu.sync_copy(w_hbm.at[pl.ds(tb, TAIL)], w_v.at[pl.ds(0, TAIL)])
        pltpu.async_copy(x_hbm.at[idxs8_v], hsrc_v.at[pl.ds(0, TAIL)], sem).wait()
        lax.fori_loop(0, TAIL, edge_body, 0)
        pltpu.sync_copy(msg_v.at[pl.ds(0, TAIL)], acc_sh.at[idxd8_v], add=True)

        plsc.subcore_barrier()

        pltpu.sync_copy(
            acc_sh.at[pl.ds(sid * ROWS_PER_TILE, ROWS_PER_TILE)],
            out_hbm.at[cid, pl.ds(sid * ROWS_PER_TILE, ROWS_PER_TILE)])

    return body(x, src, dst, wext)


def _combine_body(p_ref, o_ref):
    a = p_ref[0] + p_ref[1]
    deg = jnp.maximum(a[:, WEXT:WEXT + 1], 1.0)
    s = a[:, :NS] / deg
    v = a[:, NS:WEXT] / deg
    r = lax.broadcasted_iota(jnp.int32, (3 * NV, 3 * NV), 0)
    c = lax.broadcasted_iota(jnp.int32, (3 * NV, 3 * NV), 1)
    perm = ((c % 3) * NV + (c // 3) == r).astype(jnp.float32)
    vp = jnp.dot(v, perm, preferred_element_type=jnp.float32)
    o_ref[...] = jnp.concatenate([s, vp], axis=1)


def _combine(partials):
    grid = (10,)
    blk = N_NODES // 10
    return pl.pallas_call(
        _combine_body,
        grid=grid,
        in_specs=[pl.BlockSpec((2, blk, MROW), lambda i: (0, i, 0))],
        out_specs=pl.BlockSpec((blk, NS + 3 * NV), lambda i: (i, 0)),
        out_shape=jax.ShapeDtypeStruct((N_NODES, NS + 3 * NV), jnp.float32),
    )(partials)


def kernel(x, edge_index, edge_attr, edge_sh, W1, b1, W2, b2):
    src = edge_index[0].astype(jnp.int32)
    dst = edge_index[1].astype(jnp.int32)
    wext = _mlp(edge_attr, edge_sh, W1, b1.reshape(1, HID),
                W2, b2.reshape(1, NS + NV))
    x_pad = jnp.pad(x, ((0, 0), (0, 128 - NS)))
    partials = _sc_kernel(x_pad, src, dst, wext)
    return _combine(partials)


# speedup vs baseline: 3.1939x; 1.1898x over previous
import functools

import jax
import jax.numpy as jnp
from jax import lax
from jax.experimental import pallas as pl
from jax.experimental.pallas import tpu as pltpu
from jax.experimental.pallas import tpu_sc as plsc

NS = 64
NV = 16
HID = 192
N_NODES = 10000
N_EDGES = 160000

WEXT = NS + 3 * NV
MROW = 128
NTILES = 32
CHUNK = 64
EDGES_PER_TILE = N_EDGES // NTILES
FULL_ITERS = EDGES_PER_TILE // CHUNK
PAIRS = FULL_ITERS // 2
TAIL = EDGES_PER_TILE - FULL_ITERS * CHUNK
ACC_ROWS = 10112
ROWS_PER_TILE = ACC_ROWS // 16
MLP_BLK = 1000
XW = 128


def _mlp_body(a_ref, sh_ref, w1_ref, b1_ref, w2_ref, b2_ref, o_ref):
    h = jnp.maximum(
        jnp.dot(a_ref[...], w1_ref[...], preferred_element_type=jnp.float32)
        + b1_ref[...], 0.0)
    w = jnp.dot(h, w2_ref[...], preferred_element_type=jnp.float32) + b2_ref[...]
    ws = w[:, :NS]
    wv = w[:, NS:NS + NV]
    u0 = wv * sh_ref[:, 1:2]
    u1 = wv * sh_ref[:, 2:3]
    u2 = wv * sh_ref[:, 3:4]
    pad = jnp.zeros((MLP_BLK, MROW - WEXT), jnp.float32)
    o_ref[...] = jnp.concatenate([ws, u0, u1, u2, pad], axis=1)


def _mlp(edge_attr, edge_sh, W1, b1, W2, b2):
    grid = (N_EDGES // MLP_BLK,)
    return pl.pallas_call(
        _mlp_body,
        grid=grid,
        in_specs=[
            pl.BlockSpec((MLP_BLK, HID), lambda i: (i, 0)),
            pl.BlockSpec((MLP_BLK, 4), lambda i: (i, 0)),
            pl.BlockSpec((HID, HID), lambda i: (0, 0)),
            pl.BlockSpec((1, HID), lambda i: (0, 0)),
            pl.BlockSpec((HID, NS + NV), lambda i: (0, 0)),
            pl.BlockSpec((1, NS + NV), lambda i: (0, 0)),
        ],
        out_specs=pl.BlockSpec((MLP_BLK, MROW), lambda i: (i, 0)),
        out_shape=jax.ShapeDtypeStruct((N_EDGES, MROW), jnp.float32),
    )(edge_attr, edge_sh, W1, b1, W2, b2)


def _sc_kernel(x, src, dst, wext):
    mesh = plsc.VectorSubcoreMesh(core_axis_name="c", subcore_axis_name="s")

    @functools.partial(
        pl.kernel,
        mesh=mesh,
        out_type=jax.ShapeDtypeStruct((2, ACC_ROWS, MROW), jnp.float32),
        scratch_types=[
            pltpu.VMEM((CHUNK,), jnp.int32),
            pltpu.VMEM((CHUNK,), jnp.int32),
            pltpu.VMEM((CHUNK,), jnp.int32),
            pltpu.VMEM((CHUNK,), jnp.int32),
            pltpu.VMEM((TAIL,), jnp.int32),
            pltpu.VMEM((TAIL,), jnp.int32),
            pltpu.VMEM((CHUNK, XW), jnp.float32),
            pltpu.VMEM((CHUNK, XW), jnp.float32),
            pltpu.VMEM((CHUNK, MROW), jnp.float32),
            pltpu.VMEM((CHUNK, MROW), jnp.float32),
            pltpu.VMEM((CHUNK, MROW), jnp.float32),
            pltpu.VMEM((CHUNK, MROW), jnp.float32),
            pltpu.VMEM_SHARED((ACC_ROWS, MROW), jnp.float32),
            pltpu.SemaphoreType.DMA,
            pltpu.SemaphoreType.DMA,
            pltpu.SemaphoreType.DMA,
            pltpu.SemaphoreType.DMA,
            pltpu.SemaphoreType.DMA,
            pltpu.SemaphoreType.DMA,
            pltpu.SemaphoreType.DMA,
            pltpu.SemaphoreType.DMA,
            pltpu.SemaphoreType.DMA,
        ],
    )
    def body(x_hbm, src_hbm, dst_hbm, w_hbm, out_hbm,
             idxs0, idxs1, idxd0, idxd1, idxs8, idxd8,
             hsrc0, hsrc1, w0, w1, msg0, msg1, acc_sh,
             sem_in0, sem_in1, sem_w0, sem_w1, sem_g0, sem_g1,
             sem_s0, sem_s1, sem):
        cid = lax.axis_index("c")
        sid = lax.axis_index("s")
        idxs = (idxs0, idxs1)
        idxd = (idxd0, idxd1)
        hsrc = (hsrc0, hsrc1)
        wv = (w0, w1)
        msg = (msg0, msg1)
        sem_in = (sem_in0, sem_in1)
        sem_w = (sem_w0, sem_w1)
        sem_g = (sem_g0, sem_g1)
        sem_s = (sem_s0, sem_s1)

        zeros16 = jnp.zeros((16,), jnp.float32)

        def zero_buf(buf):
            def f(i, _):
                r = i // (MROW // 16)
                c = i % (MROW // 16)
                buf[r, pl.ds(c * 16, 16)] = zeros16
                return 0
            lax.fori_loop(0, CHUNK * (MROW // 16), f, 0)

        zero_buf(msg0)
        zero_buf(msg1)

        for k in range(ROWS_PER_TILE // CHUNK):
            pltpu.sync_copy(
                msg0, acc_sh.at[pl.ds(sid * ROWS_PER_TILE + k * CHUNK, CHUNK)])
        rem = ROWS_PER_TILE % CHUNK
        if rem:
            pltpu.sync_copy(
                msg0.at[pl.ds(0, rem)],
                acc_sh.at[pl.ds(sid * ROWS_PER_TILE
                                + (ROWS_PER_TILE // CHUNK) * CHUNK, rem)])

        ii = lax.broadcasted_iota(jnp.int32, (16,), 0)

        cvec = jnp.where(ii == 0, 1.0, 0.0).astype(jnp.float32)

        def crow(buf):
            def f(r, _):
                buf[r, pl.ds(WEXT, 16)] = cvec
                return 0
            lax.fori_loop(0, CHUNK, f, 0)

        crow(msg0)
        crow(msg1)

        plsc.subcore_barrier()

        tbase = (cid * 16 + sid) * EDGES_PER_TILE

        def make_edge_body(hs, wb, mb):
            def edge_body(e, _):
                h0 = hs[e, pl.ds(0, 16)]
                mb[e, pl.ds(0, 16)] = h0 * wb[e, pl.ds(0, 16)]
                for k in range(1, NS // 16):
                    mb[e, pl.ds(k * 16, 16)] = (
                        hs[e, pl.ds(k * 16, 16)] * wb[e, pl.ds(k * 16, 16)])
                for j in range(3):
                    off = NS + j * 16
                    mb[e, pl.ds(off, 16)] = h0 * wb[e, pl.ds(off, 16)]
                return 0
            return edge_body

        def half_step(p, c, S, prefetch):
            pltpu.make_async_copy(x_hbm.at[idxs[S]], hsrc[S], sem_g[S]).wait()
            pltpu.make_async_copy(
                w_hbm.at[pl.ds(0, CHUNK)], wv[S], sem_w[S]).wait()

            @pl.when(p > 0)
            def _():
                pltpu.make_async_copy(
                    msg[S], acc_sh.at[idxd[S]], sem_s[S]).wait()

            base = tbase + c * CHUNK
            pltpu.sync_copy(dst_hbm.at[pl.ds(base, CHUNK)], idxd[S])

            @pl.when(prefetch)
            def _():
                nbase = tbase + (c + 2) * CHUNK
                pltpu.async_copy(
                    src_hbm.at[pl.ds(nbase, CHUNK)], idxs[S], sem_in[S])

            lax.fori_loop(0, CHUNK, make_edge_body(hsrc[S], wv[S], msg[S]), 0)

            pltpu.async_copy(msg[S], acc_sh.at[idxd[S]], sem_s[S], add=True)

            @pl.when(prefetch)
            def _():
                nbase = tbase + (c + 2) * CHUNK
                pltpu.async_copy(
                    w_hbm.at[pl.ds(nbase, CHUNK)], wv[S], sem_w[S])
                pltpu.make_async_copy(
                    src_hbm.at[pl.ds(nbase, CHUNK)], idxs[S], sem_in[S]).wait()
                pltpu.async_copy(x_hbm.at[idxs[S]], hsrc[S], sem_g[S])

        def pair_body(p, _):
            half_step(p, 2 * p, 0, p < PAIRS - 1)
            half_step(p, 2 * p + 1, 1, p < PAIRS - 1)
            return 0

        for S in range(2):
            base = tbase + S * CHUNK
            pltpu.async_copy(src_hbm.at[pl.ds(base, CHUNK)], idxs[S], sem_in[S])
            pltpu.async_copy(w_hbm.at[pl.ds(base, CHUNK)], wv[S], sem_w[S])
        for S in range(2):
            pltpu.make_async_copy(
                src_hbm.at[pl.ds(tbase, CHUNK)], idxs[S], sem_in[S]).wait()
            pltpu.async_copy(x_hbm.at[idxs[S]], hsrc[S], sem_g[S])

        lax.fori_loop(0, PAIRS, pair_body, 0)

        pltpu.make_async_copy(msg[0], acc_sh.at[idxd[0]], sem_s[0]).wait()
        pltpu.make_async_copy(msg[1], acc_sh.at[idxd[1]], sem_s[1]).wait()

        tb = tbase + FULL_ITERS * CHUNK
        pltpu.sync_copy(src_hbm.at[pl.ds(tb, TAIL)], idxs8)
        pltpu.sync_copy(dst_hbm.at[pl.ds(tb, TAIL)], idxd8)
        pltpu.sync_copy(w_hbm.at[pl.ds(tb, TAIL)], w0.at[pl.ds(0, TAIL)])
        pltpu.async_copy(x_hbm.at[idxs8], hsrc0.at[pl.ds(0, TAIL)], sem).wait()
        lax.fori_loop(0, TAIL, make_edge_body(hsrc0, w0, msg0), 0)
        pltpu.sync_copy(msg0.at[pl.ds(0, TAIL)], acc_sh.at[idxd8], add=True)

        plsc.subcore_barrier()

        pltpu.sync_copy(
            acc_sh.at[pl.ds(sid * ROWS_PER_TILE, ROWS_PER_TILE)],
            out_hbm.at[cid, pl.ds(sid * ROWS_PER_TILE, ROWS_PER_TILE)])

    return body(x, src, dst, wext)


def _combine_body(p_ref, o_ref):
    a = p_ref[0] + p_ref[1]
    deg = jnp.maximum(a[:, WEXT:WEXT + 1], 1.0)
    s = a[:, :NS] / deg
    v = a[:, NS:WEXT] / deg
    r = lax.broadcasted_iota(jnp.int32, (3 * NV, 3 * NV), 0)
    c = lax.broadcasted_iota(jnp.int32, (3 * NV, 3 * NV), 1)
    perm = ((c % 3) * NV + (c // 3) == r).astype(jnp.float32)
    vp = jnp.dot(v, perm, preferred_element_type=jnp.float32)
    o_ref[...] = jnp.concatenate([s, vp], axis=1)


def _combine(partials):
    grid = (10,)
    blk = N_NODES // 10
    return pl.pallas_call(
        _combine_body,
        grid=grid,
        in_specs=[pl.BlockSpec((2, blk, MROW), lambda i: (0, i, 0))],
        out_specs=pl.BlockSpec((blk, NS + 3 * NV), lambda i: (i, 0)),
        out_shape=jax.ShapeDtypeStruct((N_NODES, NS + 3 * NV), jnp.float32),
    )(partials)


def kernel(x, edge_index, edge_attr, edge_sh, W1, b1, W2, b2):
    src = edge_index[0].astype(jnp.int32)
    dst = edge_index[1].astype(jnp.int32)
    wext = _mlp(edge_attr, edge_sh, W1, b1.reshape(1, HID),
                W2, b2.reshape(1, NS + NV))
    x_pad = jnp.pad(x, ((0, 0), (0, XW - NS)))
    partials = _sc_kernel(x_pad, src, dst, wext)
    return _combine(partials)


# speedup vs baseline: 4.6087x; 1.4430x over previous
import functools

import jax
import jax.numpy as jnp
from jax import lax
from jax.experimental import pallas as pl
from jax.experimental.pallas import tpu as pltpu
from jax.experimental.pallas import tpu_sc as plsc

NS = 64
NV = 16
HID = 192
N_NODES = 10000
N_EDGES = 160000

WEXT = NS + 3 * NV
MROW = 128
NTILES = 32
CHUNK = 64
EDGES_PER_TILE = N_EDGES // NTILES
FULL_ITERS = EDGES_PER_TILE // CHUNK
PAIRS = FULL_ITERS // 2
TAIL = EDGES_PER_TILE - FULL_ITERS * CHUNK
ACC_ROWS = 10112
ROWS_PER_TILE = ACC_ROWS // 16
MLP_BLK = 1280
XW = 128


def _mlp_body(at_ref, sht_ref, w1_ref, b1_ref, w2_ref, b2_ref, o_ref):
    h = jnp.maximum(
        lax.dot_general(at_ref[...], w1_ref[...], (((0,), (0,)), ((), ())),
                        preferred_element_type=jnp.float32) + b1_ref[...], 0.0)
    w = jnp.dot(h, w2_ref[...], preferred_element_type=jnp.float32) + b2_ref[...]
    ws = w[:, :NS]
    wv = w[:, NS:NS + NV]
    er = lax.broadcasted_iota(jnp.int32, (4, 3), 0)
    ec = lax.broadcasted_iota(jnp.int32, (4, 3), 1)
    sel = (er == ec + 1).astype(jnp.float32)
    sh3 = lax.dot_general(sht_ref[...], sel, (((0,), (0,)), ((), ())),
                          preferred_element_type=jnp.float32)
    u0 = wv * sh3[:, 0:1]
    u1 = wv * sh3[:, 1:2]
    u2 = wv * sh3[:, 2:3]
    pad = jnp.zeros((MLP_BLK, MROW - WEXT), jnp.float32)
    o_ref[...] = jnp.concatenate([ws, u0, u1, u2, pad], axis=1)


def _mlp(edge_attr_t, edge_sh_t, W1, b1, W2, b2):
    grid = (N_EDGES // MLP_BLK,)
    return pl.pallas_call(
        _mlp_body,
        grid=grid,
        in_specs=[
            pl.BlockSpec((HID, MLP_BLK), lambda i: (0, i)),
            pl.BlockSpec((4, MLP_BLK), lambda i: (0, i)),
            pl.BlockSpec((HID, HID), lambda i: (0, 0)),
            pl.BlockSpec((1, HID), lambda i: (0, 0)),
            pl.BlockSpec((HID, NS + NV), lambda i: (0, 0)),
            pl.BlockSpec((1, NS + NV), lambda i: (0, 0)),
        ],
        out_specs=pl.BlockSpec((MLP_BLK, MROW), lambda i: (i, 0)),
        out_shape=jax.ShapeDtypeStruct((N_EDGES, MROW), jnp.float32),
    )(edge_attr_t, edge_sh_t, W1, b1, W2, b2)


def _sc_kernel(x, src, dst, wext):
    mesh = plsc.VectorSubcoreMesh(core_axis_name="c", subcore_axis_name="s")

    @functools.partial(
        pl.kernel,
        mesh=mesh,
        out_type=jax.ShapeDtypeStruct((2, ACC_ROWS, MROW), jnp.float32),
        scratch_types=[
            pltpu.VMEM((CHUNK,), jnp.int32),
            pltpu.VMEM((CHUNK,), jnp.int32),
            pltpu.VMEM((CHUNK,), jnp.int32),
            pltpu.VMEM((CHUNK,), jnp.int32),
            pltpu.VMEM((TAIL,), jnp.int32),
            pltpu.VMEM((TAIL,), jnp.int32),
            pltpu.VMEM((CHUNK, XW), jnp.float32),
            pltpu.VMEM((CHUNK, XW), jnp.float32),
            pltpu.VMEM((CHUNK, MROW), jnp.float32),
            pltpu.VMEM((CHUNK, MROW), jnp.float32),
            pltpu.VMEM((CHUNK, MROW), jnp.float32),
            pltpu.VMEM((CHUNK, MROW), jnp.float32),
            pltpu.VMEM_SHARED((ACC_ROWS, MROW), jnp.float32),
            pltpu.SemaphoreType.DMA,
            pltpu.SemaphoreType.DMA,
            pltpu.SemaphoreType.DMA,
            pltpu.SemaphoreType.DMA,
            pltpu.SemaphoreType.DMA,
            pltpu.SemaphoreType.DMA,
            pltpu.SemaphoreType.DMA,
            pltpu.SemaphoreType.DMA,
            pltpu.SemaphoreType.DMA,
        ],
    )
    def body(x_hbm, src_hbm, dst_hbm, w_hbm, out_hbm,
             idxs0, idxs1, idxd0, idxd1, idxs8, idxd8,
             hsrc0, hsrc1, w0, w1, msg0, msg1, acc_sh,
             sem_in0, sem_in1, sem_w0, sem_w1, sem_g0, sem_g1,
             sem_s0, sem_s1, sem):
        cid = lax.axis_index("c")
        sid = lax.axis_index("s")
        idxs = (idxs0, idxs1)
        idxd = (idxd0, idxd1)
        hsrc = (hsrc0, hsrc1)
        wv = (w0, w1)
        msg = (msg0, msg1)
        sem_in = (sem_in0, sem_in1)
        sem_w = (sem_w0, sem_w1)
        sem_g = (sem_g0, sem_g1)
        sem_s = (sem_s0, sem_s1)

        zeros16 = jnp.zeros((16,), jnp.float32)

        def zero_buf(buf):
            def f(i, _):
                r = i // (MROW // 16)
                c = i % (MROW // 16)
                buf[r, pl.ds(c * 16, 16)] = zeros16
                return 0
            lax.fori_loop(0, CHUNK * (MROW // 16), f, 0)

        zero_buf(msg0)
        zero_buf(msg1)

        for k in range(ROWS_PER_TILE // CHUNK):
            pltpu.sync_copy(
                msg0, acc_sh.at[pl.ds(sid * ROWS_PER_TILE + k * CHUNK, CHUNK)])
        rem = ROWS_PER_TILE % CHUNK
        if rem:
            pltpu.sync_copy(
                msg0.at[pl.ds(0, rem)],
                acc_sh.at[pl.ds(sid * ROWS_PER_TILE
                                + (ROWS_PER_TILE // CHUNK) * CHUNK, rem)])

        ii = lax.broadcasted_iota(jnp.int32, (16,), 0)

        cvec = jnp.where(ii == 0, 1.0, 0.0).astype(jnp.float32)

        def crow(buf):
            def f(r, _):
                buf[r, pl.ds(WEXT, 16)] = cvec
                return 0
            lax.fori_loop(0, CHUNK, f, 0)

        crow(msg0)
        crow(msg1)

        plsc.subcore_barrier()

        tbase = (cid * 16 + sid) * EDGES_PER_TILE

        def make_edge_body(hs, wb, mb):
            def edge_body(e, _):
                h0 = hs[e, pl.ds(0, 16)]
                mb[e, pl.ds(0, 16)] = h0 * wb[e, pl.ds(0, 16)]
                for k in range(1, NS // 16):
                    mb[e, pl.ds(k * 16, 16)] = (
                        hs[e, pl.ds(k * 16, 16)] * wb[e, pl.ds(k * 16, 16)])
                for j in range(3):
                    off = NS + j * 16
                    mb[e, pl.ds(off, 16)] = h0 * wb[e, pl.ds(off, 16)]
                return 0
            return edge_body

        def half_step(p, c, S, prefetch):
            pltpu.make_async_copy(x_hbm.at[idxs[S]], hsrc[S], sem_g[S]).wait()
            pltpu.make_async_copy(
                w_hbm.at[pl.ds(0, CHUNK)], wv[S], sem_w[S]).wait()

            @pl.when(p > 0)
            def _():
                pltpu.make_async_copy(
                    msg[S], acc_sh.at[idxd[S]], sem_s[S]).wait()

            base = tbase + c * CHUNK
            pltpu.sync_copy(dst_hbm.at[pl.ds(base, CHUNK)], idxd[S])

            @pl.when(prefetch)
            def _():
                nbase = tbase + (c + 2) * CHUNK
                pltpu.async_copy(
                    src_hbm.at[pl.ds(nbase, CHUNK)], idxs[S], sem_in[S])

            lax.fori_loop(0, CHUNK, make_edge_body(hsrc[S], wv[S], msg[S]), 0)

            pltpu.async_copy(msg[S], acc_sh.at[idxd[S]], sem_s[S], add=True)

            @pl.when(prefetch)
            def _():
                nbase = tbase + (c + 2) * CHUNK
                pltpu.async_copy(
                    w_hbm.at[pl.ds(nbase, CHUNK)], wv[S], sem_w[S])
                pltpu.make_async_copy(
                    src_hbm.at[pl.ds(nbase, CHUNK)], idxs[S], sem_in[S]).wait()
                pltpu.async_copy(x_hbm.at[idxs[S]], hsrc[S], sem_g[S])

        def pair_body(p, _):
            half_step(p, 2 * p, 0, p < PAIRS - 1)
            half_step(p, 2 * p + 1, 1, p < PAIRS - 1)
            return 0

        for S in range(2):
            base = tbase + S * CHUNK
            pltpu.async_copy(src_hbm.at[pl.ds(base, CHUNK)], idxs[S], sem_in[S])
            pltpu.async_copy(w_hbm.at[pl.ds(base, CHUNK)], wv[S], sem_w[S])
        for S in range(2):
            pltpu.make_async_copy(
                src_hbm.at[pl.ds(tbase, CHUNK)], idxs[S], sem_in[S]).wait()
            pltpu.async_copy(x_hbm.at[idxs[S]], hsrc[S], sem_g[S])

        lax.fori_loop(0, PAIRS, pair_body, 0)

        pltpu.make_async_copy(msg[0], acc_sh.at[idxd[0]], sem_s[0]).wait()
        pltpu.make_async_copy(msg[1], acc_sh.at[idxd[1]], sem_s[1]).wait()

        tb = tbase + FULL_ITERS * CHUNK
        pltpu.sync_copy(src_hbm.at[pl.ds(tb, TAIL)], idxs8)
        pltpu.sync_copy(dst_hbm.at[pl.ds(tb, TAIL)], idxd8)
        pltpu.sync_copy(w_hbm.at[pl.ds(tb, TAIL)], w0.at[pl.ds(0, TAIL)])
        pltpu.async_copy(x_hbm.at[idxs8], hsrc0.at[pl.ds(0, TAIL)], sem).wait()
        lax.fori_loop(0, TAIL, make_edge_body(hsrc0, w0, msg0), 0)
        pltpu.sync_copy(msg0.at[pl.ds(0, TAIL)], acc_sh.at[idxd8], add=True)

        plsc.subcore_barrier()

        pltpu.sync_copy(
            acc_sh.at[pl.ds(sid * ROWS_PER_TILE, ROWS_PER_TILE)],
            out_hbm.at[cid, pl.ds(sid * ROWS_PER_TILE, ROWS_PER_TILE)])

    return body(x, src, dst, wext)


def _combine_body(p_ref, o_ref):
    a = p_ref[0] + p_ref[1]
    deg = jnp.maximum(a[:, WEXT:WEXT + 1], 1.0)
    s = a[:, :NS] / deg
    v = a[:, NS:WEXT] / deg
    r = lax.broadcasted_iota(jnp.int32, (3 * NV, 3 * NV), 0)
    c = lax.broadcasted_iota(jnp.int32, (3 * NV, 3 * NV), 1)
    perm = ((c % 3) * NV + (c // 3) == r).astype(jnp.float32)
    vp = jnp.dot(v, perm, preferred_element_type=jnp.float32)
    o_ref[...] = jnp.concatenate([s, vp], axis=1)


def _combine(partials):
    grid = (10,)
    blk = N_NODES // 10
    return pl.pallas_call(
        _combine_body,
        grid=grid,
        in_specs=[pl.BlockSpec((2, blk, MROW), lambda i: (0, i, 0))],
        out_specs=pl.BlockSpec((blk, NS + 3 * NV), lambda i: (i, 0)),
        out_shape=jax.ShapeDtypeStruct((N_NODES, NS + 3 * NV), jnp.float32),
    )(partials)


def kernel(x, edge_index, edge_attr, edge_sh, W1, b1, W2, b2):
    src = edge_index[0].astype(jnp.int32)
    dst = edge_index[1].astype(jnp.int32)
    wext = _mlp(edge_attr.T, edge_sh.T, W1, b1.reshape(1, HID),
                W2, b2.reshape(1, NS + NV))
    x_pad = jnp.pad(x, ((0, 0), (0, XW - NS)))
    partials = _sc_kernel(x_pad, src, dst, wext)
    return _combine(partials)
